# Initial kernel scaffold; baseline (speedup 1.0000x reference)
import jax
import jax.numpy as jnp
from jax.experimental import pallas as pl


def kernel(output, target, normalize):
    raise NotImplementedError("write your pallas kernel here")


# speedup vs baseline: 2.3757x; 2.3757x over previous
import math

import jax
import jax.numpy as jnp
from jax.experimental import pallas as pl
from jax.experimental.pallas import tpu as pltpu

_SIZE = 100000
_PAD = 0
_SV = 0.1 / (_SIZE - 2)
_CONF = 0.9
_C_ROW = (_SIZE - 2) * _SV * math.log(_SV) + _CONF * math.log(_CONF)

_BC = 2048
_GRID = (_SIZE + _BC - 1) // _BC


def _kl_kernel(out_ref, tgt_ref, acc_ref):
    j = pl.program_id(0)
    x = out_ref[...]
    t = tgt_ref[...]
    cols = j * _BC + jax.lax.broadcasted_iota(jnp.int32, x.shape, 1)
    x = jnp.where(cols < _SIZE, x, 0.0)
    w = jnp.where(cols == t, _CONF, _SV)
    w = jnp.where(cols == _PAD, 0.0, w)
    w = jnp.where(t == _PAD, 0.0, w)
    part = jnp.sum(w * x)

    @pl.when(j == 0)
    def _init():
        cnt = jnp.sum((t != _PAD).astype(jnp.float32))
        acc_ref[0, 0] = _C_ROW * cnt

    acc_ref[0, 0] -= part


def kernel(output, target, normalize):
    n = output.shape[0]
    target = target.astype(jnp.int32)
    acc = pl.pallas_call(
        _kl_kernel,
        grid=(_GRID,),
        in_specs=[
            pl.BlockSpec((n, _BC), lambda j: (0, j)),
            pl.BlockSpec((n, 1), lambda j: (0, 0)),
        ],
        out_specs=pl.BlockSpec((1, 1), lambda j: (0, 0), memory_space=pltpu.SMEM),
        out_shape=jax.ShapeDtypeStruct((1, 1), jnp.float32),
    )(output, target)
    return acc[0, 0] / jnp.asarray(normalize, dtype=jnp.float32)
